# baseline (speedup 1.0000x reference)
import jax
import jax.numpy as jnp
from jax import lax
from jax.experimental import pallas as pl
from jax.experimental.pallas import tpu as pltpu
from jax.experimental.pallas import tpu_sc as plsc

B = 4096
L = 50
EMB = 64
NC, NS = 2, 16
NW = NC * NS
SEQ_W = B // NW
NBUF = 8
AHEAD = 4


def _emb_kernel(src_tbl, tgt_tbl, src_tok, tgt_tok, out, idx_v, *scratch):
    rows = scratch[:NBUF]
    gsem = scratch[NBUF:2 * NBUF]
    wsem = scratch[2 * NBUF:]
    wid = lax.axis_index("s") * NC + lax.axis_index("c")
    seq_base = wid * SEQ_W

    for t, (tbl, tok) in enumerate(((src_tbl, src_tok), (tgt_tbl, tgt_tok))):
        pltpu.sync_copy(tok.at[pl.ds(seq_base, SEQ_W)], idx_v)

        def gather(s, b):
            pltpu.async_copy(tbl.at[idx_v.at[s]], rows[b], gsem[b])

        def gather_wait(s, b):
            pltpu.make_async_copy(tbl.at[idx_v.at[s]], rows[b],
                                  gsem[b]).wait()

        def wb(s, b):
            pltpu.async_copy(rows[b], out.at[t, seq_base + s], wsem[b])

        def wb_wait(s, b):
            pltpu.make_async_copy(rows[b], out.at[t, seq_base + s],
                                  wsem[b]).wait()

        for b in range(AHEAD):
            gather(b, b)

        @pl.loop(0, SEQ_W, step=NBUF)
        def _(s0):
            for b in range(NBUF):
                s = s0 + b
                nxt = (b + AHEAD) % NBUF

                @pl.when(s >= NBUF - AHEAD)
                def _():
                    wb_wait(s + AHEAD - NBUF, nxt)

                @pl.when(s < SEQ_W - AHEAD)
                def _():
                    gather(s + AHEAD, nxt)

                gather_wait(s, b)
                wb(s, b)

        for s in range(SEQ_W - (NBUF - AHEAD), SEQ_W):
            wb_wait(s, s % NBUF)


@jax.jit
def kernel(src_tokens, tgt_tokens, src_table, tgt_table):
    mesh = plsc.VectorSubcoreMesh(core_axis_name="c", subcore_axis_name="s")
    return pl.kernel(
        _emb_kernel,
        out_type=jax.ShapeDtypeStruct((2, B, L, EMB), jnp.float32),
        mesh=mesh,
        scratch_types=(
            [pltpu.VMEM((SEQ_W, L), jnp.int32)]
            + [pltpu.VMEM((L, EMB), jnp.float32) for _ in range(NBUF)]
            + [pltpu.SemaphoreType.DMA for _ in range(2 * NBUF)]
        ),
        compiler_params=pltpu.CompilerParams(use_tc_tiling_on_sc=False),
    )(src_table, tgt_table, src_tokens.astype(jnp.int32),
      tgt_tokens.astype(jnp.int32))
